# baseline (speedup 1.0000x reference)
import functools

import jax
import jax.numpy as jnp
from jax import lax
from jax.experimental import pallas as pl
from jax.experimental.pallas import tpu as pltpu
from jax.experimental.pallas import tpu_sc as plsc

B, C, N_IN = 4, 128, 32768
N_OUT, K = 8192, 16
D = B * C
L = 16

NC, NS = 2, 16
NW = NC * NS
ROWS_PER_W = N_OUT // NW
G = 4
NBLK = ROWS_PER_W // G


def _sc_kernel_body(table_hbm, idx_hbm, out_hbm, idx_v, rows_v, out_v, sem):
    wid = lax.axis_index("s") * NC + lax.axis_index("c")
    base = wid * ROWS_PER_W

    pltpu.sync_copy(idx_hbm.at[wid], idx_v)

    def block_body(g, carry):
        pltpu.async_copy(table_hbm.at[idx_v.at[g]], rows_v, sem).wait()

        for r in range(G):
            def dg_body(dg, c2, r=r):
                off = dg * L
                acc = rows_v[r * K, pl.ds(off, L)]
                for j in range(1, K):
                    acc = jnp.maximum(acc, rows_v[r * K + j, pl.ds(off, L)])
                out_v[r, pl.ds(off, L)] = acc
                return c2
            lax.fori_loop(0, D // L, dg_body, 0, unroll=True)

        pltpu.sync_copy(out_v, out_hbm.at[pl.ds(base + g * G, G)])
        return carry

    lax.fori_loop(0, NBLK, block_body, 0)


@jax.jit
def _max_pool_sc(table, idx_grouped):
    mesh = plsc.VectorSubcoreMesh(core_axis_name="c", subcore_axis_name="s")
    kfn = functools.partial(
        pl.kernel,
        mesh=mesh,
        out_type=jax.ShapeDtypeStruct((N_OUT, D), jnp.float32),
        scratch_types=[
            pltpu.VMEM((NBLK, G * K), jnp.int32),
            pltpu.VMEM((G * K, D), jnp.float32),
            pltpu.VMEM((G, D), jnp.float32),
            pltpu.SemaphoreType.DMA,
        ],
    )(_sc_kernel_body)
    return kfn(table, idx_grouped)


def kernel(x, idx):
    table = x.reshape(B * C, N_IN).T
    idx_grouped = idx.reshape(NW, NBLK, G * K)
    out_t = _max_pool_sc(table, idx_grouped)
    return out_t.T.reshape(B, C, N_OUT)


# speedup vs baseline: 3.0374x; 3.0374x over previous
import functools

import jax
import jax.numpy as jnp
from jax import lax
from jax.experimental import pallas as pl
from jax.experimental.pallas import tpu as pltpu
from jax.experimental.pallas import tpu_sc as plsc

B, C, N_IN = 4, 128, 32768
N_OUT, K = 8192, 16
D = B * C
L = 16

NC, NS = 2, 16
NW = NC * NS
ROWS_PER_W = N_OUT // NW
G = 4
NBLK = ROWS_PER_W // G
NBUF = 3
NITER = NBLK // NBUF + (1 if NBLK % NBUF else 0)


def _sc_kernel_body(table_hbm, idx_hbm, out_hbm, idx_v,
                    rows_bufs, out_bufs, gsems, osems):
    wid = lax.axis_index("s") * NC + lax.axis_index("c")
    base = wid * ROWS_PER_W

    pltpu.sync_copy(idx_hbm.at[wid], idx_v)

    for b in range(NBUF):
        pltpu.async_copy(table_hbm.at[idx_v.at[b]], rows_bufs.at[b],
                         gsems.at[b])

    def iter_body(i, carry):
        for b in range(NBUF):
            g = i * NBUF + b

            @pl.when(g < NBLK)
            def _(b=b, g=g):
                rows_v = rows_bufs.at[b]
                out_v = out_bufs.at[b]
                pltpu.make_async_copy(table_hbm.at[idx_v.at[g]], rows_v,
                                      gsems.at[b]).wait()
                @pl.when(i > 0)
                def _():
                    pltpu.make_async_copy(
                        out_v, out_hbm.at[pl.ds(base, G)], osems.at[b]).wait()

                for r in range(G):
                    def dg_body(dg, c2, r=r):
                        off = dg * L
                        acc = rows_v[r * K, pl.ds(off, L)]
                        for j in range(1, K):
                            acc = jnp.maximum(
                                acc, rows_v[r * K + j, pl.ds(off, L)])
                        out_v[r, pl.ds(off, L)] = acc
                        return c2
                    lax.fori_loop(0, D // L, dg_body, 0, unroll=4)

                @pl.when(g + NBUF < NBLK)
                def _():
                    pltpu.async_copy(table_hbm.at[idx_v.at[g + NBUF]],
                                     rows_v, gsems.at[b])
                pltpu.async_copy(out_v, out_hbm.at[pl.ds(base + g * G, G)],
                                 osems.at[b])
        return carry

    lax.fori_loop(0, NITER, iter_body, 0)

    for b in range(NBUF):
        pltpu.make_async_copy(out_bufs.at[b], out_hbm.at[pl.ds(base, G)],
                              osems.at[b]).wait()


@jax.jit
def _max_pool_sc(table, idx_grouped):
    mesh = plsc.VectorSubcoreMesh(core_axis_name="c", subcore_axis_name="s")
    kfn = functools.partial(
        pl.kernel,
        mesh=mesh,
        out_type=jax.ShapeDtypeStruct((N_OUT, D), jnp.float32),
        scratch_types=[
            pltpu.VMEM((NBLK, G * K), jnp.int32),
            pltpu.VMEM((NBUF, G * K, D), jnp.float32),
            pltpu.VMEM((NBUF, G, D), jnp.float32),
            pltpu.SemaphoreType.DMA((NBUF,)),
            pltpu.SemaphoreType.DMA((NBUF,)),
        ],
    )(_sc_kernel_body)
    return kfn(table, idx_grouped)


def kernel(x, idx):
    table = x.reshape(B * C, N_IN).T
    idx_grouped = idx.reshape(NW, NBLK, G * K)
    out_t = _max_pool_sc(table, idx_grouped)
    return out_t.T.reshape(B, C, N_OUT)
